# baseline (speedup 1.0000x reference)
import functools

import jax
import jax.numpy as jnp
from jax import lax
from jax.experimental import pallas as pl
from jax.experimental.pallas import tpu as pltpu
from jax.experimental.pallas import tpu_sc as plsc

NC = 2
NS = 16
NW = NC * NS
C = 128
DEGW = 128
SPLIT0 = 0.62


def _mesh():
    return plsc.VectorSubcoreMesh(
        core_axis_name="c", subcore_axis_name="s", num_cores=NC, num_subcores=NS
    )


def _make_deg_kernel(npad, calloc):
    rows_per_tile = npad // NS
    assert calloc % 4 == 0 and rows_per_tile % 128 == 0
    groups = calloc // 4

    @functools.partial(
        pl.kernel,
        out_type=jax.ShapeDtypeStruct((NC, npad, DEGW), jnp.float32),
        mesh=_mesh(),
        scratch_types=[
            pltpu.VMEM((calloc, C), jnp.int32),
            pltpu.VMEM((C, DEGW), jnp.float32),
            pltpu.VMEM((C, DEGW), jnp.float32),
            pltpu.VMEM_SHARED((npad, DEGW), jnp.float32),
            pltpu.SemaphoreType.DMA,
            pltpu.SemaphoreType.DMA,
            pltpu.SemaphoreType.DMA,
            pltpu.SemaphoreType.DMA,
        ],
    )
    def deg_kernel(dst_hbm, out_hbm, idx_v, ones_v, zeros_v, acc_sh,
                   s0, s1, s2, s3):
        sems = (s0, s1, s2, s3)
        c = lax.axis_index("c")
        s = lax.axis_index("s")
        w = c * NS + s

        def fill(i, carry):
            for kk in range(DEGW // 16):
                ones_v[i, pl.ds(kk * 16, 16)] = jnp.full((16,), 1.0, jnp.float32)
                zeros_v[i, pl.ds(kk * 16, 16)] = jnp.zeros((16,), jnp.float32)
            return carry

        lax.fori_loop(0, C, fill, 0)

        r0 = s * rows_per_tile

        def zblk(i, carry):
            pltpu.sync_copy(zeros_v, acc_sh.at[pl.ds(r0 + i * C, C)])
            return carry

        lax.fori_loop(0, rows_per_tile // C, zblk, 0)
        plsc.subcore_barrier()

        pltpu.async_copy(dst_hbm.at[w], idx_v, s0).wait()

        def body(g, carry):
            j0 = g * 4
            for b in range(4):
                pltpu.async_copy(
                    ones_v, acc_sh.at[idx_v.at[j0 + b]], sems[b], add=True
                )
            for b in range(4):
                pltpu.make_async_copy(
                    ones_v, acc_sh.at[idx_v.at[j0 + b]], sems[b]
                ).wait()
            return carry

        lax.fori_loop(0, groups, body, 0)
        plsc.subcore_barrier()

        def oblk(i, carry):
            pltpu.sync_copy(
                acc_sh.at[pl.ds(r0 + i * C, C)], out_hbm.at[c, pl.ds(r0 + i * C, C)]
            )
            return carry

        lax.fori_loop(0, rows_per_tile // C, oblk, 0)

    return deg_kernel


def _make_scatter_kernel(npad, chunks0, chunks1, calloc, d):
    rows_per_tile = npad // NS

    @functools.partial(
        pl.kernel,
        out_type=jax.ShapeDtypeStruct((NC, npad, d), jnp.float32),
        mesh=_mesh(),
        scratch_types=[
            pltpu.VMEM((calloc, C), jnp.int32),
            pltpu.VMEM((calloc, C), jnp.int32),
            pltpu.VMEM((C, d), jnp.float32),
            pltpu.VMEM_SHARED((npad, d), jnp.float32),
            pltpu.SemaphoreType.DMA,
        ],
    )
    def scatter_kernel(table_hbm, src_hbm, dst_hbm, out_hbm, src_v, dst_v, buf,
                       acc_sh, sem):
        c = lax.axis_index("c")
        s = lax.axis_index("s")
        w = c * NS + s
        nchunks = lax.select(c == 0, jnp.int32(chunks0), jnp.int32(chunks1))

        def zrow(i, carry):
            for kk in range(d // 16):
                buf[i, pl.ds(kk * 16, 16)] = jnp.zeros((16,), jnp.float32)
            return carry

        lax.fori_loop(0, C, zrow, 0)

        r0 = s * rows_per_tile

        def zblk(i, carry):
            pltpu.sync_copy(buf, acc_sh.at[pl.ds(r0 + i * C, C)])
            return carry

        lax.fori_loop(0, rows_per_tile // C, zblk, 0)
        plsc.subcore_barrier()

        pltpu.async_copy(src_hbm.at[w], src_v, sem).wait()
        pltpu.async_copy(dst_hbm.at[w], dst_v, sem).wait()

        def body(j, carry):
            pltpu.async_copy(table_hbm.at[src_v.at[j]], buf, sem).wait()
            pltpu.sync_copy(buf, acc_sh.at[dst_v.at[j]], add=True)
            return carry

        lax.fori_loop(0, nchunks, body, 0)
        plsc.subcore_barrier()

        def oblk(i, carry):
            pltpu.sync_copy(
                acc_sh.at[pl.ds(r0 + i * C, C)],
                out_hbm.at[c, pl.ds(r0 + i * C, C)],
            )
            return carry

        lax.fori_loop(0, rows_per_tile // C, oblk, 0)

    return scatter_kernel


def _elu(x):
    return jnp.where(x > 0, x, jnp.exp(x) - 1.0)


def _prep_body(x_ref, w1_ref, degp_ref, dinv_ref, hw1s_ref):
    xb = x_ref[...]
    deg = 1.0 + degp_ref[0, :, 0:1] + degp_ref[1, :, 0:1]
    dinv = lax.rsqrt(deg)
    hw = lax.dot_general(
        xb, w1_ref[...], (((0,), (0,)), ((), ())),
        preferred_element_type=jnp.float32,
    )
    hw1s_ref[...] = hw * dinv
    dinv_ref[...] = jnp.broadcast_to(dinv, dinv_ref.shape)


def _mid_body(p_ref, hw1s_ref, dinv_ref, b1_ref, w2_ref, hw2s_ref):
    acc = p_ref[0] + p_ref[1] + hw1s_ref[...]
    dinv = dinv_ref[:, 0:1]
    o = acc * dinv + b1_ref[...]
    h2 = _elu(o)
    hw2 = jnp.dot(h2, w2_ref[...], preferred_element_type=jnp.float32)
    hw2s_ref[...] = hw2 * dinv


def _fin_body(q_ref, hw2s_ref, dinv_ref, b2_ref, wfc_ref, bfc_ref, y_ref):
    acc = q_ref[0] + q_ref[1] + hw2s_ref[...]
    dinv = dinv_ref[:, 0:1]
    o = acc * dinv + b2_ref[...]
    h2 = _elu(o)
    y = jnp.dot(h2, wfc_ref[...], preferred_element_type=jnp.float32) + bfc_ref[...]
    y_ref[...] = y


def _split_edges(idx, n, chunks0, chunks1, calloc):
    e0 = NS * chunks0 * C
    cap = NS * calloc * C
    part0 = jnp.concatenate(
        [idx[:e0], jnp.full((cap - e0,), n, jnp.int32)]
    ).reshape(NS, calloc, C)
    rest = idx[e0:]
    part1 = jnp.concatenate(
        [rest, jnp.full((cap - rest.shape[0],), n, jnp.int32)]
    ).reshape(NS, calloc, C)
    return jnp.concatenate([part0, part1], axis=0)


def kernel(x, edge_index, W1, b1, W2, b2, Wfc, bfc):
    _, d, n = x.shape
    e = edge_index.shape[1]
    npad = ((n + NS * 128 - 1) // (NS * 128)) * (NS * 128)
    bn = 1024
    assert npad % bn == 0 and d % 16 == 0

    e0 = int(e * SPLIT0)
    chunks0 = (e0 + NS * C - 1) // (NS * C)
    chunks1 = (e - NS * chunks0 * C + NS * C - 1) // (NS * C)
    calloc = (max(chunks0, chunks1) + 3) // 4 * 4
    src_p = _split_edges(edge_index[0], n, chunks0, chunks1, calloc)
    dst_p = _split_edges(edge_index[1], n, chunks0, chunks1, calloc)
    x_pad = jnp.pad(x[0], ((0, 0), (0, npad - n)))

    degp = _make_deg_kernel(npad, calloc)(dst_p)

    grid = (npad // bn,)
    dinv, hw1s = pl.pallas_call(
        _prep_body,
        grid=grid,
        in_specs=[
            pl.BlockSpec((d, bn), lambda i: (0, i)),
            pl.BlockSpec((d, d), lambda i: (0, 0)),
            pl.BlockSpec((2, bn, DEGW), lambda i: (0, i, 0)),
        ],
        out_specs=[
            pl.BlockSpec((bn, 8), lambda i: (i, 0)),
            pl.BlockSpec((bn, d), lambda i: (i, 0)),
        ],
        out_shape=[
            jax.ShapeDtypeStruct((npad, 8), jnp.float32),
            jax.ShapeDtypeStruct((npad, d), jnp.float32),
        ],
    )(x_pad, W1, degp)

    p1 = _make_scatter_kernel(npad, chunks0, chunks1, calloc, d)(hw1s, src_p, dst_p)

    hw2s = pl.pallas_call(
        _mid_body,
        grid=grid,
        in_specs=[
            pl.BlockSpec((2, bn, d), lambda i: (0, i, 0)),
            pl.BlockSpec((bn, d), lambda i: (i, 0)),
            pl.BlockSpec((bn, 8), lambda i: (i, 0)),
            pl.BlockSpec((1, d), lambda i: (0, 0)),
            pl.BlockSpec((d, d), lambda i: (0, 0)),
        ],
        out_specs=pl.BlockSpec((bn, d), lambda i: (i, 0)),
        out_shape=jax.ShapeDtypeStruct((npad, d), jnp.float32),
    )(p1, hw1s, dinv, b1.reshape(1, d), W2)

    p2 = _make_scatter_kernel(npad, chunks0, chunks1, calloc, d)(hw2s, src_p, dst_p)

    y = pl.pallas_call(
        _fin_body,
        grid=grid,
        in_specs=[
            pl.BlockSpec((2, bn, d), lambda i: (0, i, 0)),
            pl.BlockSpec((bn, d), lambda i: (i, 0)),
            pl.BlockSpec((bn, 8), lambda i: (i, 0)),
            pl.BlockSpec((1, d), lambda i: (0, 0)),
            pl.BlockSpec((d, 1), lambda i: (0, 0)),
            pl.BlockSpec((1, 1), lambda i: (0, 0)),
        ],
        out_specs=pl.BlockSpec((bn, 1), lambda i: (i, 0)),
        out_shape=jax.ShapeDtypeStruct((npad, 1), jnp.float32),
    )(p2, hw2s, dinv, b2.reshape(1, d), Wfc, bfc.reshape(1, 1))

    return y[:n, 0].reshape(1, 1, 1, n)


# speedup vs baseline: 6.5978x; 6.5978x over previous
import functools

import jax
import jax.numpy as jnp
from jax import lax
from jax.experimental import pallas as pl
from jax.experimental.pallas import tpu as pltpu
from jax.experimental.pallas import tpu_sc as plsc

NC = 2
NS = 16
NW = NC * NS
C = 128
DEGW = 128
SPLIT0 = 0.62


def _mesh():
    return plsc.VectorSubcoreMesh(
        core_axis_name="c", subcore_axis_name="s", num_cores=NC, num_subcores=NS
    )


def _make_deg_kernel(npad, calloc):
    rows_per_tile = npad // NS
    assert calloc % 4 == 0 and rows_per_tile % 128 == 0
    groups = calloc // 4

    @functools.partial(
        pl.kernel,
        out_type=jax.ShapeDtypeStruct((NC, npad, DEGW), jnp.float32),
        mesh=_mesh(),
        scratch_types=[
            pltpu.VMEM((calloc, C), jnp.int32),
            pltpu.VMEM((C, DEGW), jnp.float32),
            pltpu.VMEM((C, DEGW), jnp.float32),
            pltpu.VMEM_SHARED((npad, DEGW), jnp.float32),
            pltpu.SemaphoreType.DMA,
            pltpu.SemaphoreType.DMA,
            pltpu.SemaphoreType.DMA,
            pltpu.SemaphoreType.DMA,
        ],
    )
    def deg_kernel(dst_hbm, out_hbm, idx_v, ones_v, zeros_v, acc_sh,
                   s0, s1, s2, s3):
        sems = (s0, s1, s2, s3)
        c = lax.axis_index("c")
        s = lax.axis_index("s")
        w = c * NS + s

        def fill(i, carry):
            for kk in range(DEGW // 16):
                ones_v[i, pl.ds(kk * 16, 16)] = jnp.full((16,), 1.0, jnp.float32)
                zeros_v[i, pl.ds(kk * 16, 16)] = jnp.zeros((16,), jnp.float32)
            return carry

        lax.fori_loop(0, C, fill, 0)

        r0 = s * rows_per_tile

        def zblk(i, carry):
            pltpu.sync_copy(zeros_v, acc_sh.at[pl.ds(r0 + i * C, C)])
            return carry

        lax.fori_loop(0, rows_per_tile // C, zblk, 0)
        plsc.subcore_barrier()

        pltpu.async_copy(dst_hbm.at[w], idx_v, s0).wait()

        def body(g, carry):
            j0 = g * 4
            for b in range(4):
                pltpu.async_copy(
                    ones_v, acc_sh.at[idx_v.at[j0 + b]], sems[b], add=True
                )
            for b in range(4):
                pltpu.make_async_copy(
                    ones_v, acc_sh.at[idx_v.at[j0 + b]], sems[b]
                ).wait()
            return carry

        lax.fori_loop(0, groups, body, 0)
        plsc.subcore_barrier()

        def oblk(i, carry):
            pltpu.sync_copy(
                acc_sh.at[pl.ds(r0 + i * C, C)], out_hbm.at[c, pl.ds(r0 + i * C, C)]
            )
            return carry

        lax.fori_loop(0, rows_per_tile // C, oblk, 0)

    return deg_kernel


def _make_scatter_kernel(npad, chunks0, chunks1, calloc, d):
    rows_per_tile = npad // NS

    @functools.partial(
        pl.kernel,
        out_type=jax.ShapeDtypeStruct((NC, npad, d), jnp.float32),
        mesh=_mesh(),
        scratch_types=[
            pltpu.VMEM((calloc, C), jnp.int32),
            pltpu.VMEM((calloc, C), jnp.int32),
            pltpu.VMEM((C, d), jnp.float32),
            pltpu.VMEM_SHARED((npad, d), jnp.float32),
            pltpu.SemaphoreType.DMA,
        ],
    )
    def scatter_kernel(table_hbm, src_hbm, dst_hbm, out_hbm, src_v, dst_v, buf,
                       acc_sh, sem):
        c = lax.axis_index("c")
        s = lax.axis_index("s")
        w = c * NS + s

        def zrow(i, carry):
            for kk in range(d // 16):
                buf[i, pl.ds(kk * 16, 16)] = jnp.zeros((16,), jnp.float32)
            return carry

        lax.fori_loop(0, C, zrow, 0)

        r0 = s * rows_per_tile

        def zblk(i, carry):
            pltpu.sync_copy(buf, acc_sh.at[pl.ds(r0 + i * C, C)])
            return carry

        lax.fori_loop(0, rows_per_tile // C, zblk, 0)
        plsc.subcore_barrier()

        pltpu.async_copy(src_hbm.at[w], src_v, sem).wait()
        pltpu.async_copy(dst_hbm.at[w], dst_v, sem).wait()

        def body(j, carry):
            pltpu.async_copy(table_hbm.at[src_v.at[j]], buf, sem).wait()
            pltpu.sync_copy(buf, acc_sh.at[dst_v.at[j]], add=True)
            return carry

        @pl.when(c == 0)
        def _loop0():
            lax.fori_loop(0, chunks0, body, 0)

        @pl.when(c != 0)
        def _loop1():
            lax.fori_loop(0, chunks1, body, 0)

        plsc.subcore_barrier()

        def oblk(i, carry):
            pltpu.sync_copy(
                acc_sh.at[pl.ds(r0 + i * C, C)],
                out_hbm.at[c, pl.ds(r0 + i * C, C)],
            )
            return carry

        lax.fori_loop(0, rows_per_tile // C, oblk, 0)

    return scatter_kernel


def _elu(x):
    return jnp.where(x > 0, x, jnp.exp(x) - 1.0)


def _prep_body(x_ref, w1_ref, degp_ref, dinv_ref, hw1s_ref):
    xb = x_ref[...]
    deg = 1.0 + degp_ref[0, :, 0:1] + degp_ref[1, :, 0:1]
    dinv = lax.rsqrt(deg)
    hw = lax.dot_general(
        xb, w1_ref[...], (((0,), (0,)), ((), ())),
        preferred_element_type=jnp.float32,
    )
    hw1s_ref[...] = hw * dinv
    dinv_ref[...] = jnp.broadcast_to(dinv, dinv_ref.shape)


def _mid_body(p_ref, hw1s_ref, dinv_ref, b1_ref, w2_ref, hw2s_ref):
    acc = p_ref[0] + p_ref[1] + hw1s_ref[...]
    dinv = dinv_ref[:, 0:1]
    o = acc * dinv + b1_ref[...]
    h2 = _elu(o)
    hw2 = jnp.dot(h2, w2_ref[...], preferred_element_type=jnp.float32)
    hw2s_ref[...] = hw2 * dinv


def _fin_body(q_ref, hw2s_ref, dinv_ref, b2_ref, wfc_ref, bfc_ref, y_ref):
    acc = q_ref[0] + q_ref[1] + hw2s_ref[...]
    dinv = dinv_ref[:, 0:1]
    o = acc * dinv + b2_ref[...]
    h2 = _elu(o)
    y = jnp.dot(h2, wfc_ref[...], preferred_element_type=jnp.float32) + bfc_ref[...]
    y_ref[...] = y


def _split_edges(idx, n, chunks0, chunks1, calloc):
    m0 = NS * chunks0 * C

    def part(sl, nchunks):
        filled = jnp.concatenate(
            [sl, jnp.full((NS * nchunks * C - sl.shape[0],), n, jnp.int32)]
        ).reshape(NS, nchunks, C)
        tail = jnp.full((NS, calloc - nchunks, C), n, jnp.int32)
        return jnp.concatenate([filled, tail], axis=1)

    return jnp.concatenate(
        [part(idx[:m0], chunks0), part(idx[m0:], chunks1)], axis=0
    )


def kernel(x, edge_index, W1, b1, W2, b2, Wfc, bfc):
    _, d, n = x.shape
    e = edge_index.shape[1]
    npad = ((n + NS * 128 - 1) // (NS * 128)) * (NS * 128)
    bn = 1024
    assert npad % bn == 0 and d % 16 == 0

    e0 = int(e * SPLIT0)
    chunks0 = (e0 + NS * C - 1) // (NS * C)
    chunks1 = (e - NS * chunks0 * C + NS * C - 1) // (NS * C)
    calloc = (max(chunks0, chunks1) + 3) // 4 * 4
    src_p = _split_edges(edge_index[0], n, chunks0, chunks1, calloc)
    dst_p = _split_edges(edge_index[1], n, chunks0, chunks1, calloc)
    x_pad = jnp.pad(x[0], ((0, 0), (0, npad - n)))

    degp = _make_deg_kernel(npad, calloc)(dst_p)

    grid = (npad // bn,)
    dinv, hw1s = pl.pallas_call(
        _prep_body,
        grid=grid,
        in_specs=[
            pl.BlockSpec((d, bn), lambda i: (0, i)),
            pl.BlockSpec((d, d), lambda i: (0, 0)),
            pl.BlockSpec((2, bn, DEGW), lambda i: (0, i, 0)),
        ],
        out_specs=[
            pl.BlockSpec((bn, 8), lambda i: (i, 0)),
            pl.BlockSpec((bn, d), lambda i: (i, 0)),
        ],
        out_shape=[
            jax.ShapeDtypeStruct((npad, 8), jnp.float32),
            jax.ShapeDtypeStruct((npad, d), jnp.float32),
        ],
    )(x_pad, W1, degp)

    p1 = _make_scatter_kernel(npad, chunks0, chunks1, calloc, d)(hw1s, src_p, dst_p)

    hw2s = pl.pallas_call(
        _mid_body,
        grid=grid,
        in_specs=[
            pl.BlockSpec((2, bn, d), lambda i: (0, i, 0)),
            pl.BlockSpec((bn, d), lambda i: (i, 0)),
            pl.BlockSpec((bn, 8), lambda i: (i, 0)),
            pl.BlockSpec((1, d), lambda i: (0, 0)),
            pl.BlockSpec((d, d), lambda i: (0, 0)),
        ],
        out_specs=pl.BlockSpec((bn, d), lambda i: (i, 0)),
        out_shape=jax.ShapeDtypeStruct((npad, d), jnp.float32),
    )(p1, hw1s, dinv, b1.reshape(1, d), W2)

    p2 = _make_scatter_kernel(npad, chunks0, chunks1, calloc, d)(hw2s, src_p, dst_p)

    y = pl.pallas_call(
        _fin_body,
        grid=grid,
        in_specs=[
            pl.BlockSpec((2, bn, d), lambda i: (0, i, 0)),
            pl.BlockSpec((bn, d), lambda i: (i, 0)),
            pl.BlockSpec((bn, 8), lambda i: (i, 0)),
            pl.BlockSpec((1, d), lambda i: (0, 0)),
            pl.BlockSpec((d, 1), lambda i: (0, 0)),
            pl.BlockSpec((1, 1), lambda i: (0, 0)),
        ],
        out_specs=pl.BlockSpec((bn, 1), lambda i: (i, 0)),
        out_shape=jax.ShapeDtypeStruct((npad, 1), jnp.float32),
    )(p2, hw2s, dinv, b2.reshape(1, d), Wfc, bfc.reshape(1, 1))

    return y[:n, 0].reshape(1, 1, 1, n)


# speedup vs baseline: 7.3341x; 1.1116x over previous
import functools

import jax
import jax.numpy as jnp
from jax import lax
from jax.experimental import pallas as pl
from jax.experimental.pallas import tpu as pltpu
from jax.experimental.pallas import tpu_sc as plsc

NC = 2
NS = 16
NW = NC * NS
C = 128
DEGW = 128
SPLIT0 = 0.62


def _mesh():
    return plsc.VectorSubcoreMesh(
        core_axis_name="c", subcore_axis_name="s", num_cores=NC, num_subcores=NS
    )


def _make_deg_kernel(npad, chunks0, chunks1, calloc):
    rows_per_tile = npad // NS
    assert rows_per_tile % 128 == 0

    @functools.partial(
        pl.kernel,
        out_type=jax.ShapeDtypeStruct((NC, npad, DEGW), jnp.float32),
        mesh=_mesh(),
        scratch_types=[
            pltpu.VMEM((calloc, C), jnp.int32),
            pltpu.VMEM((C, DEGW), jnp.float32),
            pltpu.VMEM((C, DEGW), jnp.float32),
            pltpu.VMEM_SHARED((npad, DEGW), jnp.float32),
            pltpu.SemaphoreType.DMA,
            pltpu.SemaphoreType.DMA,
            pltpu.SemaphoreType.DMA,
            pltpu.SemaphoreType.DMA,
        ],
    )
    def deg_kernel(dst_hbm, out_hbm, idx_v, ones_v, zeros_v, acc_sh,
                   s0, s1, s2, s3):
        sems = (s0, s1, s2, s3)
        c = lax.axis_index("c")
        s = lax.axis_index("s")
        w = c * NS + s

        def fill(i, carry):
            for kk in range(DEGW // 16):
                ones_v[i, pl.ds(kk * 16, 16)] = jnp.full((16,), 1.0, jnp.float32)
                zeros_v[i, pl.ds(kk * 16, 16)] = jnp.zeros((16,), jnp.float32)
            return carry

        lax.fori_loop(0, C, fill, 0)

        r0 = s * rows_per_tile

        def zblk(i, carry):
            pltpu.sync_copy(zeros_v, acc_sh.at[pl.ds(r0 + i * C, C)])
            return carry

        lax.fori_loop(0, rows_per_tile // C, zblk, 0)
        plsc.subcore_barrier()

        pltpu.async_copy(dst_hbm.at[w], idx_v, s0).wait()

        def body(j, carry):
            pltpu.sync_copy(ones_v, acc_sh.at[idx_v.at[j]], add=True)
            return carry

        @pl.when(c == 0)
        def _loop0():
            lax.fori_loop(0, chunks0, body, 0)

        @pl.when(c != 0)
        def _loop1():
            lax.fori_loop(0, chunks1, body, 0)

        plsc.subcore_barrier()

        def oblk(i, carry):
            pltpu.sync_copy(
                acc_sh.at[pl.ds(r0 + i * C, C)], out_hbm.at[c, pl.ds(r0 + i * C, C)]
            )
            return carry

        lax.fori_loop(0, rows_per_tile // C, oblk, 0)

    return deg_kernel


def _make_scatter_kernel(npad, chunks0, chunks1, calloc, d):
    rows_per_tile = npad // NS

    @functools.partial(
        pl.kernel,
        out_type=jax.ShapeDtypeStruct((NC, npad, d), jnp.float32),
        mesh=_mesh(),
        scratch_types=[
            pltpu.VMEM((calloc, C), jnp.int32),
            pltpu.VMEM((calloc, C), jnp.int32),
            pltpu.VMEM((C, d), jnp.float32),
            pltpu.VMEM_SHARED((npad, d), jnp.float32),
            pltpu.SemaphoreType.DMA,
        ],
    )
    def scatter_kernel(table_hbm, src_hbm, dst_hbm, out_hbm, src_v, dst_v, buf,
                       acc_sh, sem):
        c = lax.axis_index("c")
        s = lax.axis_index("s")
        w = c * NS + s

        def zrow(i, carry):
            for kk in range(d // 16):
                buf[i, pl.ds(kk * 16, 16)] = jnp.zeros((16,), jnp.float32)
            return carry

        lax.fori_loop(0, C, zrow, 0)

        r0 = s * rows_per_tile

        def zblk(i, carry):
            pltpu.sync_copy(buf, acc_sh.at[pl.ds(r0 + i * C, C)])
            return carry

        lax.fori_loop(0, rows_per_tile // C, zblk, 0)
        plsc.subcore_barrier()

        pltpu.async_copy(src_hbm.at[w], src_v, sem).wait()
        pltpu.async_copy(dst_hbm.at[w], dst_v, sem).wait()

        def body(j, carry):
            pltpu.async_copy(table_hbm.at[src_v.at[j]], buf, sem).wait()
            pltpu.sync_copy(buf, acc_sh.at[dst_v.at[j]], add=True)
            return carry

        @pl.when(c == 0)
        def _loop0():
            lax.fori_loop(0, chunks0, body, 0)

        @pl.when(c != 0)
        def _loop1():
            lax.fori_loop(0, chunks1, body, 0)

        plsc.subcore_barrier()

        def oblk(i, carry):
            pltpu.sync_copy(
                acc_sh.at[pl.ds(r0 + i * C, C)],
                out_hbm.at[c, pl.ds(r0 + i * C, C)],
            )
            return carry

        lax.fori_loop(0, rows_per_tile // C, oblk, 0)

    return scatter_kernel


def _elu(x):
    return jnp.where(x > 0, x, jnp.exp(x) - 1.0)


def _prep_body(x_ref, w1_ref, degp_ref, dinv_ref, hw1s_ref):
    xb = x_ref[...]
    deg = 1.0 + degp_ref[0, :, 0:1] + degp_ref[1, :, 0:1]
    dinv = lax.rsqrt(deg)
    hw = lax.dot_general(
        xb, w1_ref[...], (((0,), (0,)), ((), ())),
        preferred_element_type=jnp.float32,
    )
    hw1s_ref[...] = hw * dinv
    dinv_ref[...] = jnp.broadcast_to(dinv, dinv_ref.shape)


def _mid_body(p_ref, hw1s_ref, dinv_ref, b1_ref, w2_ref, hw2s_ref):
    acc = p_ref[0] + p_ref[1] + hw1s_ref[...]
    dinv = dinv_ref[:, 0:1]
    o = acc * dinv + b1_ref[...]
    h2 = _elu(o)
    hw2 = jnp.dot(h2, w2_ref[...], preferred_element_type=jnp.float32)
    hw2s_ref[...] = hw2 * dinv


def _fin_body(q_ref, hw2s_ref, dinv_ref, b2_ref, wfc_ref, bfc_ref, y_ref):
    acc = q_ref[0] + q_ref[1] + hw2s_ref[...]
    dinv = dinv_ref[:, 0:1]
    o = acc * dinv + b2_ref[...]
    h2 = _elu(o)
    y = jnp.dot(h2, wfc_ref[...], preferred_element_type=jnp.float32) + bfc_ref[...]
    y_ref[...] = y


def _split_edges(idx, n, chunks0, chunks1, calloc):
    m0 = NS * chunks0 * C

    def part(sl, nchunks):
        filled = jnp.concatenate(
            [sl, jnp.full((NS * nchunks * C - sl.shape[0],), n, jnp.int32)]
        ).reshape(NS, nchunks, C)
        tail = jnp.full((NS, calloc - nchunks, C), n, jnp.int32)
        return jnp.concatenate([filled, tail], axis=1)

    return jnp.concatenate(
        [part(idx[:m0], chunks0), part(idx[m0:], chunks1)], axis=0
    )


def kernel(x, edge_index, W1, b1, W2, b2, Wfc, bfc):
    _, d, n = x.shape
    e = edge_index.shape[1]
    npad = ((n + NS * 128 - 1) // (NS * 128)) * (NS * 128)
    bn = 1024
    assert npad % bn == 0 and d % 16 == 0

    e0 = int(e * SPLIT0)
    chunks0 = (e0 + NS * C - 1) // (NS * C)
    chunks1 = (e - NS * chunks0 * C + NS * C - 1) // (NS * C)
    calloc = max(chunks0, chunks1)
    src_p = _split_edges(edge_index[0], n, chunks0, chunks1, calloc)
    dst_p = _split_edges(edge_index[1], n, chunks0, chunks1, calloc)
    x_pad = jnp.pad(x[0], ((0, 0), (0, npad - n)))

    degp = _make_deg_kernel(npad, chunks0, chunks1, calloc)(dst_p)

    grid = (npad // bn,)
    dinv, hw1s = pl.pallas_call(
        _prep_body,
        grid=grid,
        in_specs=[
            pl.BlockSpec((d, bn), lambda i: (0, i)),
            pl.BlockSpec((d, d), lambda i: (0, 0)),
            pl.BlockSpec((2, bn, DEGW), lambda i: (0, i, 0)),
        ],
        out_specs=[
            pl.BlockSpec((bn, 8), lambda i: (i, 0)),
            pl.BlockSpec((bn, d), lambda i: (i, 0)),
        ],
        out_shape=[
            jax.ShapeDtypeStruct((npad, 8), jnp.float32),
            jax.ShapeDtypeStruct((npad, d), jnp.float32),
        ],
    )(x_pad, W1, degp)

    p1 = _make_scatter_kernel(npad, chunks0, chunks1, calloc, d)(hw1s, src_p, dst_p)

    hw2s = pl.pallas_call(
        _mid_body,
        grid=grid,
        in_specs=[
            pl.BlockSpec((2, bn, d), lambda i: (0, i, 0)),
            pl.BlockSpec((bn, d), lambda i: (i, 0)),
            pl.BlockSpec((bn, 8), lambda i: (i, 0)),
            pl.BlockSpec((1, d), lambda i: (0, 0)),
            pl.BlockSpec((d, d), lambda i: (0, 0)),
        ],
        out_specs=pl.BlockSpec((bn, d), lambda i: (i, 0)),
        out_shape=jax.ShapeDtypeStruct((npad, d), jnp.float32),
    )(p1, hw1s, dinv, b1.reshape(1, d), W2)

    p2 = _make_scatter_kernel(npad, chunks0, chunks1, calloc, d)(hw2s, src_p, dst_p)

    y = pl.pallas_call(
        _fin_body,
        grid=grid,
        in_specs=[
            pl.BlockSpec((2, bn, d), lambda i: (0, i, 0)),
            pl.BlockSpec((bn, d), lambda i: (i, 0)),
            pl.BlockSpec((bn, 8), lambda i: (i, 0)),
            pl.BlockSpec((1, d), lambda i: (0, 0)),
            pl.BlockSpec((d, 1), lambda i: (0, 0)),
            pl.BlockSpec((1, 1), lambda i: (0, 0)),
        ],
        out_specs=pl.BlockSpec((bn, 1), lambda i: (i, 0)),
        out_shape=jax.ShapeDtypeStruct((npad, 1), jnp.float32),
    )(p2, hw2s, dinv, b2.reshape(1, d), Wfc, bfc.reshape(1, 1))

    return y[:n, 0].reshape(1, 1, 1, n)
